# Initial kernel scaffold; baseline (speedup 1.0000x reference)
import jax
import jax.numpy as jnp
from jax.experimental import pallas as pl


def kernel(x, Ae, Ad, bd, lambda_pre):
    raise NotImplementedError("write your pallas kernel here")


# speedup vs baseline: 13.9125x; 13.9125x over previous
import functools

import jax
import jax.numpy as jnp
from jax.experimental import pallas as pl
from jax.experimental.pallas import tpu as pltpu

NTOK = 2048
DIMIN = 768
WIDTH = 16384
KVAL = 64

RB = 256
WT = 2048
N_RB = NTOK // RB
N_WT = WIDTH // WT
N_BISECT = 26


def _encode_body(x_ref, ae_ref, z_ref, t_ref, zbuf):
    j = pl.program_id(1)
    zj = jax.lax.dot_general(
        x_ref[...], ae_ref[...],
        dimension_numbers=(((1,), (1,)), ((), ())),
        preferred_element_type=jnp.float32,
    )
    zbuf[:, pl.ds(j * WT, WT)] = zj
    z_ref[...] = zj

    @pl.when(j == N_WT - 1)
    def _():
        zb = zbuf[...]
        hi = jnp.max(zb, axis=1, keepdims=True)
        hi = jnp.maximum(hi, 1e-30)
        lo = jnp.zeros_like(hi)

        def body(_, carry):
            lo, hi = carry
            mid = 0.5 * (lo + hi)
            cnt = jnp.sum((zb > mid).astype(jnp.float32), axis=1, keepdims=True)
            pred = cnt >= KVAL
            return jnp.where(pred, mid, lo), jnp.where(pred, hi, mid)

        lo, hi = jax.lax.fori_loop(0, N_BISECT, body, (lo, hi))
        t_ref[...] = lo


def _decode_body(z_ref, t_ref, ad_ref, lam_ref, out_ref):
    j = pl.program_id(1)

    @pl.when(j == 0)
    def _():
        out_ref[...] = jnp.zeros_like(out_ref)

    z = z_ref[...]
    t = t_ref[...]
    lam = lam_ref[0, 0]
    codes = jnp.where(z > t, z * lam, 0.0).astype(jnp.bfloat16)
    out_ref[...] += jax.lax.dot_general(
        codes, ad_ref[...],
        dimension_numbers=(((1,), (0,)), ((), ())),
        preferred_element_type=jnp.float32,
    )


def kernel(x, Ae, Ad, bd, lambda_pre):
    lam = jax.nn.softplus(lambda_pre).reshape(1, 1).astype(jnp.float32)
    xb = (x - bd).astype(jnp.bfloat16)
    aeb = Ae.astype(jnp.bfloat16)
    adb = Ad.T.astype(jnp.bfloat16)

    z, t = pl.pallas_call(
        _encode_body,
        grid=(N_RB, N_WT),
        in_specs=[
            pl.BlockSpec((RB, DIMIN), lambda i, j: (i, 0)),
            pl.BlockSpec((WT, DIMIN), lambda i, j: (j, 0)),
        ],
        out_specs=[
            pl.BlockSpec((RB, WT), lambda i, j: (i, j)),
            pl.BlockSpec((RB, 1), lambda i, j: (i, 0)),
        ],
        out_shape=[
            jax.ShapeDtypeStruct((NTOK, WIDTH), jnp.float32),
            jax.ShapeDtypeStruct((NTOK, 1), jnp.float32),
        ],
        scratch_shapes=[pltpu.VMEM((RB, WIDTH), jnp.float32)],
    )(xb, aeb)

    out = pl.pallas_call(
        _decode_body,
        grid=(N_RB, N_WT),
        in_specs=[
            pl.BlockSpec((RB, WT), lambda i, j: (i, j)),
            pl.BlockSpec((RB, 1), lambda i, j: (i, 0)),
            pl.BlockSpec((WT, DIMIN), lambda i, j: (j, 0)),
            pl.BlockSpec((1, 1), lambda i, j: (0, 0), memory_space=pltpu.SMEM),
        ],
        out_specs=pl.BlockSpec((RB, DIMIN), lambda i, j: (i, 0)),
        out_shape=jax.ShapeDtypeStruct((NTOK, DIMIN), jnp.float32),
    )(z, t, adb, lam)

    return out + bd
